# baseline (speedup 1.0000x reference)
import jax
import jax.numpy as jnp
from jax import lax
from jax.experimental import pallas as pl
from jax.experimental.pallas import tpu as pltpu

_HIDDEN = 2048
_E = 64
_T = 8192
_BT = 1024
_NBLK = _T // _BT
_TOL = 1e-4
_EPS = 1e-8


def _matmul_kernel(x_ref, wt_ref, logits_ref):
    logits_ref[...] = jnp.dot(
        x_ref[...], wt_ref[...], preferred_element_type=jnp.float32
    )


def _sinkhorn_kernel(logits_ref, scores_ref, idx_ref):
    logits = logits_ref[...]
    cost = jnp.exp(logits)

    def cond(state):
        return state[2] > _TOL

    def body(state):
        d1c, _, _ = state
        t0 = jnp.sum(d1c * cost, axis=1, keepdims=True)
        d0n = (1.0 / _T) * (1.0 / (t0 + _EPS))
        s1 = jnp.sum(d0n * cost, axis=0, keepdims=True)
        d1n = (1.0 / _E) * (1.0 / (s1 + _EPS))
        err = jnp.mean(jnp.abs(d1c - d1n))
        return (d1n, d0n, err)

    init = (
        jnp.ones((1, _E), dtype=jnp.float32),
        jnp.zeros((_T, 1), dtype=jnp.float32),
        jnp.float32(1e9),
    )
    d1f, d0f, _ = lax.while_loop(cond, body, init)

    norm = (d1f * cost) * d0f
    mx = jnp.max(norm, axis=1, keepdims=True)
    iota = lax.broadcasted_iota(jnp.int32, (_T, _E), 1)
    idx = jnp.min(jnp.where(norm == mx, iota, _E), axis=1, keepdims=True)
    act = jax.nn.sigmoid(logits)
    scores_ref[...] = jnp.sum(
        jnp.where(iota == idx, act, 0.0), axis=1, keepdims=True
    )
    idx_ref[...] = idx


def kernel(x, W):
    x2 = x.reshape(-1, x.shape[-1])
    wt = W.T
    logits = pl.pallas_call(
        _matmul_kernel,
        grid=(_NBLK,),
        in_specs=[
            pl.BlockSpec((_BT, _HIDDEN), lambda i: (i, 0)),
            pl.BlockSpec((_HIDDEN, _E), lambda i: (0, 0)),
        ],
        out_specs=pl.BlockSpec((_BT, _E), lambda i: (i, 0)),
        out_shape=jax.ShapeDtypeStruct((_T, _E), jnp.float32),
        compiler_params=pltpu.CompilerParams(
            dimension_semantics=("parallel",),
        ),
    )(x2, wt)
    scores, idx = pl.pallas_call(
        _sinkhorn_kernel,
        out_shape=[
            jax.ShapeDtypeStruct((_T, 1), jnp.float32),
            jax.ShapeDtypeStruct((_T, 1), jnp.int32),
        ],
    )(logits)
    return (scores, idx)


# speedup vs baseline: 2.9070x; 2.9070x over previous
import jax
import jax.numpy as jnp
from jax import lax
from jax.experimental import pallas as pl
from jax.experimental.pallas import tpu as pltpu

_HIDDEN = 2048
_E = 64
_SEQ = 4096
_MBS = 2
_T = _SEQ * _MBS
_BS = 512
_BT = _BS * _MBS
_NBLK = _SEQ // _BS
_TOL = 1e-4
_EPS = 1e-8


def _router_kernel(x_ref, wt_ref, scores_ref, idx_ref, logits_ref):
    i = pl.program_id(0)
    xb = x_ref[...].reshape(-1, _HIDDEN)
    logits_ref[pl.ds(i * _BT, _BT), :] = jnp.dot(
        xb, wt_ref[...], preferred_element_type=jnp.float32
    )

    @pl.when(i == _NBLK - 1)
    def _finish():
        logits = logits_ref[...]
        cost = jnp.exp(logits)

        def cond(state):
            return state[2] > _TOL

        def body(state):
            d1c, _, _ = state
            t0 = jnp.sum(d1c * cost, axis=1, keepdims=True)
            d0n = (1.0 / _T) * (1.0 / (t0 + _EPS))
            s1 = jnp.sum(d0n * cost, axis=0, keepdims=True)
            d1n = (1.0 / _E) * (1.0 / (s1 + _EPS))
            err = jnp.mean(jnp.abs(d1c - d1n))
            return (d1n, d0n, err)

        init = (
            jnp.ones((1, _E), dtype=jnp.float32),
            jnp.zeros((_T, 1), dtype=jnp.float32),
            jnp.float32(1e9),
        )
        d1f, d0f, _ = lax.while_loop(cond, body, init)

        norm = (d1f * cost) * d0f
        mx = jnp.max(norm, axis=1, keepdims=True)
        iota = lax.broadcasted_iota(jnp.int32, (_T, _E), 1)
        idx = jnp.min(jnp.where(norm == mx, iota, _E), axis=1, keepdims=True)
        act = jax.nn.sigmoid(logits)
        scores_ref[...] = jnp.sum(
            jnp.where(iota == idx, act, 0.0), axis=1, keepdims=True
        )
        idx_ref[...] = idx


def kernel(x, W):
    wt = W.T
    scores, idx = pl.pallas_call(
        _router_kernel,
        grid=(_NBLK,),
        in_specs=[
            pl.BlockSpec((_BS, _MBS, _HIDDEN), lambda i: (i, 0, 0)),
            pl.BlockSpec((_HIDDEN, _E), lambda i: (0, 0)),
        ],
        out_specs=[
            pl.BlockSpec((_T, 1), lambda i: (0, 0)),
            pl.BlockSpec((_T, 1), lambda i: (0, 0)),
        ],
        out_shape=[
            jax.ShapeDtypeStruct((_T, 1), jnp.float32),
            jax.ShapeDtypeStruct((_T, 1), jnp.int32),
        ],
        scratch_shapes=[pltpu.VMEM((_T, _E), jnp.float32)],
        compiler_params=pltpu.CompilerParams(
            dimension_semantics=("arbitrary",),
        ),
    )(x, wt)
    return (scores, idx)


# speedup vs baseline: 3.2098x; 1.1042x over previous
import jax
import jax.numpy as jnp
from jax import lax
from jax.experimental import pallas as pl
from jax.experimental.pallas import tpu as pltpu

_HIDDEN = 2048
_E = 64
_SEQ = 4096
_MBS = 2
_T = _SEQ * _MBS
_BS = 512
_BT = _BS * _MBS
_NBLK = _SEQ // _BS
_TOL = 1e-4
_EPS = 1e-8


def _router_kernel(x_ref, wt_ref, scores_ref, idx_ref, lt_ref, ct_ref):
    i = pl.program_id(0)
    xb = x_ref[...].reshape(-1, _HIDDEN)
    res = jnp.dot(xb, wt_ref[...], preferred_element_type=jnp.float32)
    res_t = res.T
    lt_ref[:, pl.ds(i * _BT, _BT)] = res_t
    ct_ref[:, pl.ds(i * _BT, _BT)] = jnp.exp(res_t)

    @pl.when(i == _NBLK - 1)
    def _finish():
        logits_t = lt_ref[...]
        cost_t = ct_ref[...]

        def cond(state):
            return state[2] > _TOL

        def body(state):
            d1c, _, _ = state
            t0 = jnp.sum(d1c * cost_t, axis=0, keepdims=True)
            d0n = (1.0 / _T) * (1.0 / (t0 + _EPS))
            s1 = jnp.sum(d0n * cost_t, axis=1, keepdims=True)
            d1n = (1.0 / _E) * (1.0 / (s1 + _EPS))
            err = jnp.mean(jnp.abs(d1c - d1n))
            return (d1n, d0n, err)

        init = (
            jnp.ones((_E, 1), dtype=jnp.float32),
            jnp.zeros((1, _T), dtype=jnp.float32),
            jnp.float32(1e9),
        )
        d1f, d0f, _ = lax.while_loop(cond, body, init)

        norm_t = (d1f * cost_t) * d0f
        mx = jnp.max(norm_t, axis=0, keepdims=True)
        iota = lax.broadcasted_iota(jnp.int32, (_E, _T), 0)
        idx = jnp.min(jnp.where(norm_t == mx, iota, _E), axis=0, keepdims=True)
        sel_logit = jnp.sum(
            jnp.where(iota == idx, logits_t, 0.0), axis=0, keepdims=True
        )
        scores = jax.nn.sigmoid(sel_logit)
        scores_ref[...] = scores.reshape(_T, 1)
        idx_ref[...] = idx.reshape(_T, 1)


def kernel(x, W):
    wt = W.T
    scores, idx = pl.pallas_call(
        _router_kernel,
        grid=(_NBLK,),
        in_specs=[
            pl.BlockSpec((_BS, _MBS, _HIDDEN), lambda i: (i, 0, 0)),
            pl.BlockSpec((_HIDDEN, _E), lambda i: (0, 0)),
        ],
        out_specs=[
            pl.BlockSpec((_T, 1), lambda i: (0, 0)),
            pl.BlockSpec((_T, 1), lambda i: (0, 0)),
        ],
        out_shape=[
            jax.ShapeDtypeStruct((_T, 1), jnp.float32),
            jax.ShapeDtypeStruct((_T, 1), jnp.int32),
        ],
        scratch_shapes=[
            pltpu.VMEM((_E, _T), jnp.float32),
            pltpu.VMEM((_E, _T), jnp.float32),
        ],
        compiler_params=pltpu.CompilerParams(
            dimension_semantics=("arbitrary",),
        ),
    )(x, wt)
    return (scores, idx)


# speedup vs baseline: 3.4762x; 1.0830x over previous
import jax
import jax.numpy as jnp
from jax import lax
from jax.experimental import pallas as pl
from jax.experimental.pallas import tpu as pltpu

_HIDDEN = 2048
_E = 64
_SEQ = 4096
_MBS = 2
_T = _SEQ * _MBS
_BS = 512
_BT = _BS * _MBS
_NBLK = _SEQ // _BS
_TOL = 1e-4
_EPS = 1e-8


def _router_kernel(x_ref, w_ref, scores_ref, idx_ref, lt_ref, ct_ref):
    i = pl.program_id(0)
    xb = x_ref[...].reshape(-1, _HIDDEN)
    res = lax.dot_general(
        xb, w_ref[...], (((1,), (1,)), ((), ())),
        preferred_element_type=jnp.float32,
    )
    res_t = res.T
    lt_ref[:, pl.ds(i * _BT, _BT)] = res_t
    ct_ref[:, pl.ds(i * _BT, _BT)] = jnp.exp(res_t)

    @pl.when(i == _NBLK - 1)
    def _finish():
        logits_t = lt_ref[...]
        cost_t = ct_ref[...]

        def cond(state):
            return state[2] > _TOL

        def body(state):
            d1c, _, _ = state
            t0 = jnp.sum(d1c * cost_t, axis=0, keepdims=True)
            d0n = (1.0 / _T) * (1.0 / (t0 + _EPS))
            s1 = jnp.sum(d0n * cost_t, axis=1, keepdims=True)
            d1n = (1.0 / _E) * (1.0 / (s1 + _EPS))
            err = jnp.mean(jnp.abs(d1c - d1n))
            return (d1n, d0n, err)

        init = (
            jnp.ones((_E, 1), dtype=jnp.float32),
            jnp.zeros((1, _T), dtype=jnp.float32),
            jnp.float32(1e9),
        )
        d1f, d0f, _ = lax.while_loop(cond, body, init)

        norm_t = (d1f * cost_t) * d0f
        mx = jnp.max(norm_t, axis=0, keepdims=True)
        iota = lax.broadcasted_iota(jnp.int32, (_E, _T), 0)
        idx = jnp.min(jnp.where(norm_t == mx, iota, _E), axis=0, keepdims=True)
        sel_logit = jnp.sum(
            jnp.where(iota == idx, logits_t, 0.0), axis=0, keepdims=True
        )
        scores = jax.nn.sigmoid(sel_logit)
        scores_ref[...] = scores.reshape(_T, 1)
        idx_ref[...] = idx.reshape(_T, 1)


def kernel(x, W):
    scores, idx = pl.pallas_call(
        _router_kernel,
        grid=(_NBLK,),
        in_specs=[
            pl.BlockSpec((_BS, _MBS, _HIDDEN), lambda i: (i, 0, 0)),
            pl.BlockSpec((_E, _HIDDEN), lambda i: (0, 0)),
        ],
        out_specs=[
            pl.BlockSpec((_T, 1), lambda i: (0, 0)),
            pl.BlockSpec((_T, 1), lambda i: (0, 0)),
        ],
        out_shape=[
            jax.ShapeDtypeStruct((_T, 1), jnp.float32),
            jax.ShapeDtypeStruct((_T, 1), jnp.int32),
        ],
        scratch_shapes=[
            pltpu.VMEM((_E, _T), jnp.float32),
            pltpu.VMEM((_E, _T), jnp.float32),
        ],
        compiler_params=pltpu.CompilerParams(
            dimension_semantics=("arbitrary",),
        ),
    )(x, W)
    return (scores, idx)


# speedup vs baseline: 3.4819x; 1.0016x over previous
import jax
import jax.numpy as jnp
from jax import lax
from jax.experimental import pallas as pl
from jax.experimental.pallas import tpu as pltpu

_HIDDEN = 2048
_E = 64
_SEQ = 4096
_MBS = 2
_T = _SEQ * _MBS
_BS = 512
_BT = _BS * _MBS
_NBLK = _SEQ // _BS
_TOL = 1e-4
_EPS = 1e-8


def _router_kernel(x_ref, w_ref, scores_ref, idx_ref, lt_ref, ct_ref,
                   d0a_ref, s1a_ref):
    i = pl.program_id(0)
    xb = x_ref[...].reshape(-1, _HIDDEN)
    res = lax.dot_general(
        xb, w_ref[...], (((1,), (1,)), ((), ())),
        preferred_element_type=jnp.float32,
    )
    res_t = res.T
    cb = jnp.exp(res_t)
    lt_ref[:, pl.ds(i * _BT, _BT)] = res_t
    ct_ref[:, pl.ds(i * _BT, _BT)] = cb

    t0b = jnp.sum(cb, axis=0, keepdims=True)
    d0b = (1.0 / _T) * (1.0 / (t0b + _EPS))
    d0a_ref[:, pl.ds(i * _BT, _BT)] = d0b
    s1b = jnp.sum(d0b * cb, axis=1, keepdims=True)

    @pl.when(i == 0)
    def _init_acc():
        s1a_ref[...] = jnp.zeros((_E, 1), jnp.float32)

    s1a_ref[...] += s1b

    @pl.when(i == _NBLK - 1)
    def _finish():
        logits_t = lt_ref[...]
        cost_t = ct_ref[...]

        d1_1 = (1.0 / _E) * (1.0 / (s1a_ref[...] + _EPS))
        err_1 = jnp.mean(jnp.abs(jnp.float32(1.0) - d1_1))

        def cond(state):
            return state[2] > _TOL

        def body(state):
            d1c, _, _ = state
            t0 = jnp.sum(d1c * cost_t, axis=0, keepdims=True)
            d0n = (1.0 / _T) * (1.0 / (t0 + _EPS))
            s1 = jnp.sum(d0n * cost_t, axis=1, keepdims=True)
            d1n = (1.0 / _E) * (1.0 / (s1 + _EPS))
            err = jnp.mean(jnp.abs(d1c - d1n))
            return (d1n, d0n, err)

        init = (d1_1, d0a_ref[...], err_1)
        d1f, d0f, _ = lax.while_loop(cond, body, init)

        norm_t = (d1f * cost_t) * d0f
        mx = jnp.max(norm_t, axis=0, keepdims=True)
        iota = lax.broadcasted_iota(jnp.int32, (_E, _T), 0)
        idx = jnp.min(jnp.where(norm_t == mx, iota, _E), axis=0, keepdims=True)
        sel_logit = jnp.sum(
            jnp.where(iota == idx, logits_t, 0.0), axis=0, keepdims=True
        )
        scores = jax.nn.sigmoid(sel_logit)
        scores_ref[...] = scores.reshape(_T, 1)
        idx_ref[...] = idx.reshape(_T, 1)


def kernel(x, W):
    scores, idx = pl.pallas_call(
        _router_kernel,
        grid=(_NBLK,),
        in_specs=[
            pl.BlockSpec((_BS, _MBS, _HIDDEN), lambda i: (i, 0, 0)),
            pl.BlockSpec((_E, _HIDDEN), lambda i: (0, 0)),
        ],
        out_specs=[
            pl.BlockSpec((_T, 1), lambda i: (0, 0)),
            pl.BlockSpec((_T, 1), lambda i: (0, 0)),
        ],
        out_shape=[
            jax.ShapeDtypeStruct((_T, 1), jnp.float32),
            jax.ShapeDtypeStruct((_T, 1), jnp.int32),
        ],
        scratch_shapes=[
            pltpu.VMEM((_E, _T), jnp.float32),
            pltpu.VMEM((_E, _T), jnp.float32),
            pltpu.VMEM((1, _T), jnp.float32),
            pltpu.VMEM((_E, 1), jnp.float32),
        ],
        compiler_params=pltpu.CompilerParams(
            dimension_semantics=("arbitrary",),
        ),
    )(x, W)
    return (scores, idx)
